# baseline (speedup 1.0000x reference)
import functools

import jax
import jax.numpy as jnp
from jax import lax
from jax.experimental import pallas as pl
from jax.experimental.pallas import tpu as pltpu
from jax.experimental.pallas import tpu_sc as plsc

_B = 16384
_D = 16

_info = plsc.get_sparse_core_info()
_NC = _info.num_cores
_NS = _info.num_subcores
_L = _info.num_lanes
_NW = _NC * _NS
_PER_W = _B // _NW
_GROUPS = _PER_W // _L


def _make_sc_kernel():
    mesh = plsc.VectorSubcoreMesh(core_axis_name="c", subcore_axis_name="s")

    @functools.partial(
        pl.kernel,
        mesh=mesh,
        out_type=jax.ShapeDtypeStruct((_B,), jnp.float32),
        compiler_params=pltpu.CompilerParams(
            needs_layout_passes=False, use_tc_tiling_on_sc=False),
        scratch_types=[
            pltpu.VMEM((_PER_W,), jnp.int32),
            pltpu.VMEM((_PER_W,), jnp.int32),
            pltpu.VMEM((_PER_W, _D), jnp.float32),
            pltpu.VMEM((_PER_W, _D), jnp.float32),
            pltpu.VMEM((2 * _L,), jnp.float32),
            pltpu.VMEM((_PER_W,), jnp.float32),
            pltpu.SemaphoreType.DMA,
            pltpu.SemaphoreType.DMA,
        ],
    )
    def ncf_kernel(uidx_hbm, iidx_hbm, utab_hbm, itab_hbm, wb_hbm, out_hbm,
                   uidx_v, iidx_v, urows_v, irows_v, wb_v, out_v,
                   sem_u, sem_i):
        wid = lax.axis_index("s") * _NC + lax.axis_index("c")
        base = wid * _PER_W
        pltpu.sync_copy(uidx_hbm.at[pl.ds(base, _PER_W)], uidx_v)
        pltpu.sync_copy(iidx_hbm.at[pl.ds(base, _PER_W)], iidx_v)
        cp_u = pltpu.async_copy(utab_hbm.at[uidx_v], urows_v, sem_u)
        cp_i = pltpu.async_copy(itab_hbm.at[iidx_v], irows_v, sem_i)
        pltpu.sync_copy(wb_hbm, wb_v)
        cp_u.wait()
        cp_i.wait()

        lane = lax.iota(jnp.int32, _L)
        wvec = wb_v[pl.ds(0, _L)]
        bvec = wb_v[pl.ds(_L, _L)]

        def group_body(g, carry):
            rows = g * _L + lane
            acc = bvec
            for d in range(_D):
                col = jnp.full((_L,), d, jnp.int32)
                uv = plsc.load_gather(urows_v, [rows, col])
                iv = plsc.load_gather(irows_v, [rows, col])
                acc = acc + (uv * iv) * wvec[d]
            out_v[pl.ds(g * _L, _L)] = 1.0 / (1.0 + jnp.exp(-acc))
            return carry

        lax.fori_loop(0, _GROUPS, group_body, 0)
        pltpu.sync_copy(out_v, out_hbm.at[pl.ds(base, _PER_W)])

    return ncf_kernel


_ncf_kernel = _make_sc_kernel()


def kernel(user_input, item_input, user_table, item_table, W, b):
    uidx = user_input.reshape(_B).astype(jnp.int32)
    iidx = item_input.reshape(_B).astype(jnp.int32)
    wb = jnp.concatenate(
        [W.reshape(_D), jnp.broadcast_to(b.astype(jnp.float32), (_L,))])
    out = _ncf_kernel(uidx, iidx, user_table, item_table, wb)
    return out.reshape(_B, 1)
